# baseline (speedup 1.0000x reference)
import functools

import jax
import jax.numpy as jnp
from jax import lax
from jax.experimental import pallas as pl
from jax.experimental.pallas import tpu as pltpu
from jax.experimental.pallas import tpu_sc as plsc

NUM_CORES = 2
NUM_SUBCORES = 16
NW = NUM_CORES * NUM_SUBCORES

F = 26
V = 100000
D = 64
B = 4096
B_W = B // NW

_mesh = plsc.VectorSubcoreMesh(core_axis_name="c", subcore_axis_name="s")


@functools.partial(
    pl.kernel,
    mesh=_mesh,
    compiler_params=pltpu.CompilerParams(use_tc_tiling_on_sc=False),
    out_type=jax.ShapeDtypeStruct((B, F, D), jnp.float32),
    scratch_types=[
        pltpu.VMEM((F, B_W), jnp.int32),
        pltpu.VMEM((B_W, D), jnp.float32),
        pltpu.VMEM((B_W, D), jnp.float32),
        pltpu.SemaphoreType.DMA,
        pltpu.SemaphoreType.DMA,
    ],
)
def _sc_gather(xt_hbm, tab_hbm, out_hbm, idx_v, buf0, buf1, gsem0, gsem1):
    wid = lax.axis_index("s") * NUM_CORES + lax.axis_index("c")
    base = wid * B_W

    pltpu.sync_copy(xt_hbm.at[:, pl.ds(base, B_W)], idx_v)

    def gather_start(f, buf, sem):
        pltpu.make_async_copy(tab_hbm.at[f].at[idx_v.at[f]], buf, sem).start()

    def gather_wait(f, buf, sem):
        pltpu.make_async_copy(tab_hbm.at[f].at[idx_v.at[f]], buf, sem).wait()

    def scatter(f, buf):
        pltpu.sync_copy(buf, out_hbm.at[pl.ds(base, B_W), f])

    gather_start(0, buf0, gsem0)
    gather_start(1, buf1, gsem1)

    def loop_body(i, _):
        for b, (buf, sem) in enumerate(((buf0, gsem0), (buf1, gsem1))):
            f = 2 * i + b
            gather_wait(f, buf, sem)
            scatter(f, buf)
            gather_start(f + 2, buf, sem)
        return 0

    lax.fori_loop(0, (F - 2) // 2, loop_body, 0)

    for b, (buf, sem) in enumerate(((buf0, gsem0), (buf1, gsem1))):
        f = F - 2 + b
        gather_wait(f, buf, sem)
        scatter(f, buf)


def kernel(X, tables):
    xt = jnp.asarray(X, jnp.int32).T
    return _sc_gather(xt, tables)
